# baseline (speedup 1.0000x reference)
import functools

import jax
import jax.numpy as jnp
from jax import lax
from jax.experimental import pallas as pl
from jax.experimental.pallas import tpu as pltpu
from jax.experimental.pallas import tpu_sc as plsc

B, L, D = 16384, 50, 128
NC, NS = 2, 16
NW = NC * NS
BPW = B // NW
CB = 4
CLEN = CB * L
NCHUNK = BPW // CB
NBUF = 3


_mesh = plsc.VectorSubcoreMesh(core_axis_name="c", subcore_axis_name="s")


@functools.partial(
    pl.kernel,
    out_type=jax.ShapeDtypeStruct((B, L, D), jnp.float32),
    mesh=_mesh,
    scratch_types=[
        pltpu.VMEM((2, D), jnp.float32),
        pltpu.VMEM((BPW * L + 16,), jnp.int32),
        pltpu.VMEM((NBUF, CLEN, D), jnp.float32),
        pltpu.SemaphoreType.DMA,
        pltpu.SemaphoreType.DMA((NBUF,)),
    ],
    compiler_params=pltpu.CompilerParams(use_tc_tiling_on_sc=True,
                                         needs_layout_passes=False),
)
def _sc_lookup(idx_hbm, table_hbm, out_hbm, table_v, idx_v, rows_v,
               tsem, osems):
    wid = lax.axis_index("s") * NC + lax.axis_index("c")
    b0 = wid * BPW
    pltpu.async_copy(table_hbm, table_v, tsem).wait()
    pltpu.sync_copy(idx_hbm.at[pl.ds(b0 * L, BPW * L)],
                    idx_v.at[pl.ds(0, BPW * L)])

    def chunk_body(chunk, carry):
        slot = lax.rem(chunk, NBUF)

        @pl.when(chunk >= NBUF)
        def _():
            for q in range(CB):
                pltpu.make_async_copy(
                    rows_v.at[slot].at[pl.ds(q * L, L)],
                    out_hbm.at[b0],
                    osems.at[slot],
                ).wait()

        lane_j = [jnp.full((16,), j, jnp.int32) for j in range(16)]
        t0 = [table_v[0, pl.ds(k * 16, 16)] for k in range(D // 16)]
        t1 = [table_v[1, pl.ds(k * 16, 16)] for k in range(D // 16)]

        def rows_16(g, nrows):
            iv = idx_v[pl.ds(chunk * CLEN + g, 16)]
            for j in range(nrows):
                splat = lax.gather(
                    iv, lane_j[j][:, None],
                    lax.GatherDimensionNumbers(
                        offset_dims=(), collapsed_slice_dims=(0,),
                        start_index_map=(0,)),
                    (1,), mode=lax.GatherScatterMode.PROMISE_IN_BOUNDS)
                m = splat != 0
                for k in range(D // 16):
                    rows_v[slot, g + j, pl.ds(k * 16, 16)] = (
                        jnp.where(m, t1[k], t0[k]))

        for q in range(CB):
            for g in range(L // 16):
                rows_16(q * L + g * 16, 16)
            rows_16(q * L + (L // 16) * 16, L % 16)
            pltpu.make_async_copy(
                rows_v.at[slot].at[pl.ds(q * L, L)],
                out_hbm.at[b0 + chunk * CB + q],
                osems.at[slot],
            ).start()
        return carry

    lax.fori_loop(0, NCHUNK, chunk_body, 0)
    for k in range(NBUF):
        for q in range(CB):
            pltpu.make_async_copy(
                rows_v.at[k].at[pl.ds(q * L, L)],
                out_hbm.at[b0],
                osems.at[k],
            ).wait()


def kernel(inputs, table):
    return _sc_lookup(inputs.reshape(B * L), table)


# speedup vs baseline: 1.3004x; 1.3004x over previous
import functools

import jax
import jax.numpy as jnp
from jax import lax
from jax.experimental import pallas as pl
from jax.experimental.pallas import tpu as pltpu
from jax.experimental.pallas import tpu_sc as plsc

B, L, D = 16384, 50, 128
NC, NS = 2, 16
NW = NC * NS
BPW = B // NW
CB = 4
CLEN = CB * L
NCHUNK = BPW // CB
NBUF = 3


_mesh = plsc.VectorSubcoreMesh(core_axis_name="c", subcore_axis_name="s")


@functools.partial(
    pl.kernel,
    out_type=jax.ShapeDtypeStruct((B, L, D), jnp.float32),
    mesh=_mesh,
    scratch_types=[
        pltpu.VMEM((2, D), jnp.float32),
        pltpu.VMEM((BPW * L + 16,), jnp.int32),
        pltpu.VMEM((NBUF, CLEN, D), jnp.float32),
        pltpu.SemaphoreType.DMA,
        pltpu.SemaphoreType.DMA((NBUF,)),
    ],
    compiler_params=pltpu.CompilerParams(use_tc_tiling_on_sc=True,
                                         needs_layout_passes=False),
)
def _sc_lookup(idx_hbm, table_hbm, out_hbm, table_v, idx_v, rows_v,
               tsem, osems):
    wid = lax.axis_index("s") * NC + lax.axis_index("c")
    b0 = wid * BPW
    pltpu.async_copy(table_hbm, table_v, tsem).wait()
    pltpu.sync_copy(idx_hbm.at[pl.ds(b0 * L, BPW * L)],
                    idx_v.at[pl.ds(0, BPW * L)])

    def chunk_body(chunk, carry):
        slot = lax.rem(chunk, NBUF)

        @pl.when(chunk >= NBUF)
        def _():
            for q in range(CB):
                pltpu.make_async_copy(
                    rows_v.at[slot].at[pl.ds(q * L, L)],
                    out_hbm.at[b0],
                    osems.at[slot],
                ).wait()

        lane_j = [jnp.full((16,), j, jnp.int32) for j in range(16)]
        t0 = [table_v[0, pl.ds(k * 16, 16)] for k in range(D // 16)]
        t1 = [table_v[1, pl.ds(k * 16, 16)] for k in range(D // 16)]

        def rows_16(g, nrows):
            iv = idx_v[pl.ds(chunk * CLEN + g, 16)]
            for j in range(nrows):
                splat = lax.gather(
                    iv, lane_j[j][:, None],
                    lax.GatherDimensionNumbers(
                        offset_dims=(), collapsed_slice_dims=(0,),
                        start_index_map=(0,)),
                    (1,), mode=lax.GatherScatterMode.PROMISE_IN_BOUNDS)
                m = splat != 0
                for k in range(D // 16):
                    rows_v[slot, g + j, pl.ds(k * 16, 16)] = (
                        jnp.where(m, t1[k], t0[k]))

        def row_body(it, carry2):
            rows_16(it * 16, 16)
            return carry2

        lax.fori_loop(0, CLEN // 16, row_body, 0)
        if CLEN % 16:
            rows_16((CLEN // 16) * 16, CLEN % 16)

        for q in range(CB):
            pltpu.make_async_copy(
                rows_v.at[slot].at[pl.ds(q * L, L)],
                out_hbm.at[b0 + chunk * CB + q],
                osems.at[slot],
            ).start()
        return carry

    lax.fori_loop(0, NCHUNK, chunk_body, 0)
    for k in range(NBUF):
        for q in range(CB):
            pltpu.make_async_copy(
                rows_v.at[k].at[pl.ds(q * L, L)],
                out_hbm.at[b0],
                osems.at[k],
            ).wait()


def kernel(inputs, table):
    return _sc_lookup(inputs.reshape(B * L), table)
